# Initial kernel scaffold; baseline (speedup 1.0000x reference)
import jax
import jax.numpy as jnp
from jax.experimental import pallas as pl


def kernel(x):
    raise NotImplementedError("write your pallas kernel here")


# speedup vs baseline: 4.1223x; 4.1223x over previous
import functools

import jax
import jax.numpy as jnp
from jax import lax
from jax.experimental import pallas as pl
from jax.experimental.pallas import tpu as pltpu
from jax.experimental.pallas import tpu_sc as plsc

_N_PART = 8388608
_NDIM = 6
_N_BINS = 256
_LO, _HI = -6.0, 6.0
_BIN_W = (_HI - _LO) / _N_BINS
_INV_W = 1.0 / _BIN_W
_NOISE_SCALE = 0.05
_SEED = 0

_NC, _NS, _L = 2, 16, 16
_NW = _NC * _NS
_PER_W = _N_PART // _NW
_CHUNK = 8192
_NCHUNK = _PER_W // _CHUNK
_CW = _CHUNK * _NDIM
_HC = _N_BINS + 1
_HW = _L * _HC

_mesh = plsc.VectorSubcoreMesh(core_axis_name="c", subcore_axis_name="s")


@functools.partial(
    pl.kernel,
    out_type=jax.ShapeDtypeStruct((_NW, _HW), jnp.float32),
    mesh=_mesh,
    compiler_params=pltpu.CompilerParams(
        needs_layout_passes=False,
        use_tc_tiling_on_sc=False,
    ),
    scratch_types=[
        pltpu.VMEM((_CW,), jnp.float32),
        pltpu.VMEM((_CW,), jnp.float32),
        pltpu.VMEM((_HW,), jnp.float32),
        pltpu.SemaphoreType.DMA,
        pltpu.SemaphoreType.DMA,
    ],
)
def _hist_sc(xf, zeros_hbm, out, buf0, buf1, hist, sem0, sem1):
    cid = lax.axis_index("c")
    sid = lax.axis_index("s")
    wid = sid * _NC + cid
    base = wid * (_PER_W * _NDIM)

    pltpu.sync_copy(zeros_hbm, hist)

    lane = lax.iota(jnp.int32, _L)
    pos0 = lane * _NDIM
    laneoff = (lane * _HC).astype(jnp.float32)

    bufs = (buf0, buf1)
    sems = (sem0, sem1)

    def start(ci, b):
        pltpu.async_copy(xf.at[pl.ds(base + ci * _CW, _CW)], bufs[b], sems[b])

    def wait(b):
        pltpu.make_async_copy(xf.at[pl.ds(base, _CW)], bufs[b], sems[b]).wait()

    def process(b):
        buf = bufs[b]

        def body(it, pos):
            v = plsc.load_gather(buf, [pos])
            t = v * _INV_W + (-_LO * _INV_W)
            u = jnp.minimum(jnp.maximum(t, 0.0), float(_N_BINS - 1)) + laneoff
            i0 = u.astype(jnp.int32)
            f = u - i0.astype(jnp.float32)
            plsc.addupdate_scatter(hist, [i0], 1.0 - f)
            plsc.addupdate_scatter(hist, [i0 + 1], f)
            return pos + _L * _NDIM

        lax.fori_loop(0, _CHUNK // _L, body, pos0, unroll=8)

    start(0, 0)
    start(1, 1)

    def outer(g, carry):
        for b in range(2):
            wait(b)
            process(b)

            @pl.when(g * 2 + b + 2 < _NCHUNK)
            def _():
                start(g * 2 + b + 2, b)

        return carry

    lax.fori_loop(0, _NCHUNK // 2, outer, 0)

    pltpu.sync_copy(hist, out.at[wid])


def _finish_body(parts_ref, scale_ref, o_ref):
    s = jnp.sum(parts_ref[...], axis=0)
    o_ref[...] = jnp.maximum(s * scale_ref[...], 0.0)


_finish = pl.pallas_call(
    _finish_body,
    out_shape=jax.ShapeDtypeStruct((_HC,), jnp.float32),
)


def kernel(x):
    xf = x.reshape(-1)
    zeros = jnp.zeros((_HW,), jnp.float32)
    parts = _hist_sc(xf, zeros)
    noise = (
        jax.random.normal(jax.random.key(_SEED), (_N_BINS,), jnp.float32)
        * _NOISE_SCALE
    )
    scale = jnp.concatenate(
        [(1.0 + noise) / (_N_PART * _BIN_W), jnp.zeros((1,), jnp.float32)]
    )
    out = _finish(parts.reshape(_NW * _L, _HC), scale)
    return out[:_N_BINS]


# speedup vs baseline: 44.3084x; 10.7484x over previous
import functools

import jax
import jax.numpy as jnp
from jax import lax
from jax.experimental import pallas as pl
from jax.experimental.pallas import tpu as pltpu
from jax.experimental.pallas import tpu_sc as plsc

_N_PART = 8388608
_N_BINS = 256
_LO, _HI = -6.0, 6.0
_BIN_W = (_HI - _LO) / _N_BINS
_INV_W = 1.0 / _BIN_W
_NOISE_SCALE = 0.05
_SEED = 0

_NC, _NS, _L = 2, 16, 16
_NW = _NC * _NS
_PER_W = _N_PART // _NW
_CHUNK = 32768
_NCHUNK = _PER_W // _CHUNK
_HC = _N_BINS + 1
_HW = _L * _HC

_mesh = plsc.VectorSubcoreMesh(core_axis_name="c", subcore_axis_name="s")


@functools.partial(
    pl.kernel,
    out_type=jax.ShapeDtypeStruct((_NW, _HW), jnp.float32),
    mesh=_mesh,
    compiler_params=pltpu.CompilerParams(
        needs_layout_passes=False,
        use_tc_tiling_on_sc=False,
    ),
    scratch_types=[
        pltpu.VMEM((_CHUNK,), jnp.float32),
        pltpu.VMEM((_CHUNK,), jnp.float32),
        pltpu.VMEM((_HW,), jnp.float32),
        pltpu.SemaphoreType.DMA,
        pltpu.SemaphoreType.DMA,
    ],
)
def _hist_sc(xp, zeros_hbm, out, buf0, buf1, hist, sem0, sem1):
    cid = lax.axis_index("c")
    sid = lax.axis_index("s")
    wid = sid * _NC + cid
    base = wid * _PER_W

    pltpu.sync_copy(zeros_hbm, hist)

    lane = lax.iota(jnp.int32, _L)
    laneoff = (lane * _HC).astype(jnp.float32)

    bufs = (buf0, buf1)
    sems = (sem0, sem1)

    def start(ci, b):
        pltpu.async_copy(
            xp.at[pl.ds(base + ci * _CHUNK, _CHUNK)], bufs[b], sems[b]
        )

    def wait(b):
        pltpu.make_async_copy(
            xp.at[pl.ds(base, _CHUNK)], bufs[b], sems[b]
        ).wait()

    def process(b):
        buf = bufs[b]

        def body(it, carry):
            v = buf[pl.ds(it * _L, _L)]
            t = v * _INV_W + (-_LO * _INV_W)
            u = jnp.minimum(jnp.maximum(t, 0.0), float(_N_BINS - 1)) + laneoff
            i0 = u.astype(jnp.int32)
            f = u - i0.astype(jnp.float32)
            plsc.addupdate_scatter(hist, [i0], 1.0 - f)
            plsc.addupdate_scatter(hist, [i0 + 1], f)
            return carry

        lax.fori_loop(0, _CHUNK // _L, body, 0, unroll=8)

    start(0, 0)
    start(1, 1)

    def outer(g, carry):
        for b in range(2):
            wait(b)
            process(b)

            @pl.when(g * 2 + b + 2 < _NCHUNK)
            def _():
                start(g * 2 + b + 2, b)

        return carry

    lax.fori_loop(0, _NCHUNK // 2, outer, 0)

    pltpu.sync_copy(hist, out.at[wid])


def _finish_body(parts_ref, scale_ref, o_ref):
    s = jnp.sum(parts_ref[...], axis=0)
    o_ref[...] = jnp.maximum(s * scale_ref[...], 0.0)


_finish = pl.pallas_call(
    _finish_body,
    out_shape=jax.ShapeDtypeStruct((_HC,), jnp.float32),
)


def kernel(x):
    xp = x[:, 0]
    zeros = jnp.zeros((_HW,), jnp.float32)
    parts = _hist_sc(xp, zeros)
    noise = (
        jax.random.normal(jax.random.key(_SEED), (_N_BINS,), jnp.float32)
        * _NOISE_SCALE
    )
    scale = jnp.concatenate(
        [(1.0 + noise) / (_N_PART * _BIN_W), jnp.zeros((1,), jnp.float32)]
    )
    out = _finish(parts.reshape(_NW * _L, _HC), scale)
    return out[:_N_BINS]


# speedup vs baseline: 97.4591x; 2.1996x over previous
import functools

import jax
import jax.numpy as jnp
from jax import lax
from jax.experimental import pallas as pl
from jax.experimental.pallas import tpu as pltpu
from jax.experimental.pallas import tpu_sc as plsc

_N_PART = 8388608
_N_BINS = 256
_LO, _HI = -6.0, 6.0
_BIN_W = (_HI - _LO) / _N_BINS
_INV_W = 1.0 / _BIN_W
_NOISE_SCALE = 0.05
_SEED = 0

_NC, _NS, _L = 2, 16, 16
_NW = _NC * _NS
_PER_W = _N_PART // _NW
_CHUNK = 32768
_NCHUNK = _PER_W // _CHUNK
_HC = _N_BINS + 1
_HW = _L * _HC

_mesh = plsc.VectorSubcoreMesh(core_axis_name="c", subcore_axis_name="s")


@functools.partial(
    pl.kernel,
    out_type=jax.ShapeDtypeStruct((_NW, _HW), jnp.float32),
    mesh=_mesh,
    compiler_params=pltpu.CompilerParams(
        needs_layout_passes=False,
        use_tc_tiling_on_sc=False,
    ),
    scratch_types=[
        pltpu.VMEM((_CHUNK,), jnp.float32),
        pltpu.VMEM((_CHUNK,), jnp.float32),
        pltpu.VMEM((_HW,), jnp.float32),
        pltpu.SemaphoreType.DMA,
        pltpu.SemaphoreType.DMA,
    ],
)
def _hist_sc(xp, zeros_hbm, out, buf0, buf1, hist, sem0, sem1):
    cid = lax.axis_index("c")
    sid = lax.axis_index("s")
    wid = sid * _NC + cid
    base = wid * _PER_W

    pltpu.sync_copy(zeros_hbm, hist)

    lane = lax.iota(jnp.int32, _L)
    laneoff = (lane * _HC).astype(jnp.float32)

    bufs = (buf0, buf1)
    sems = (sem0, sem1)

    def start(ci, b):
        pltpu.async_copy(
            xp.at[pl.ds(base + ci * _CHUNK, _CHUNK)], bufs[b], sems[b]
        )

    def wait(b):
        pltpu.make_async_copy(
            xp.at[pl.ds(base, _CHUNK)], bufs[b], sems[b]
        ).wait()

    def process(b):
        buf = bufs[b]

        @plsc.parallel_loop(0, _CHUNK // _L, unroll=8)
        def body(it):
            v = buf[pl.ds(it * _L, _L)]
            t = v * _INV_W + (-_LO * _INV_W)
            u = jnp.minimum(jnp.maximum(t, 0.0), float(_N_BINS - 1)) + laneoff
            i0 = u.astype(jnp.int32)
            f = u - i0.astype(jnp.float32)
            plsc.addupdate_scatter(hist, [i0], 1.0 - f)
            plsc.addupdate_scatter(hist, [i0 + 1], f)

    start(0, 0)
    start(1, 1)

    def outer(g, carry):
        for b in range(2):
            wait(b)
            process(b)

            @pl.when(g * 2 + b + 2 < _NCHUNK)
            def _():
                start(g * 2 + b + 2, b)

        return carry

    lax.fori_loop(0, _NCHUNK // 2, outer, 0)

    pltpu.sync_copy(hist, out.at[wid])


def _finish_body(parts_ref, scale_ref, o_ref):
    s = jnp.sum(parts_ref[...], axis=0)
    o_ref[...] = jnp.maximum(s * scale_ref[...], 0.0)


_finish = pl.pallas_call(
    _finish_body,
    out_shape=jax.ShapeDtypeStruct((_HC,), jnp.float32),
)


def kernel(x):
    xp = x[:, 0]
    zeros = jnp.zeros((_HW,), jnp.float32)
    parts = _hist_sc(xp, zeros)
    noise = (
        jax.random.normal(jax.random.key(_SEED), (_N_BINS,), jnp.float32)
        * _NOISE_SCALE
    )
    scale = jnp.concatenate(
        [(1.0 + noise) / (_N_PART * _BIN_W), jnp.zeros((1,), jnp.float32)]
    )
    out = _finish(parts.reshape(_NW * _L, _HC), scale)
    return out[:_N_BINS]


# speedup vs baseline: 198.3564x; 2.0353x over previous
import functools

import jax
import jax.numpy as jnp
from jax import lax
from jax.experimental import pallas as pl
from jax.experimental.pallas import tpu as pltpu
from jax.experimental.pallas import tpu_sc as plsc

_N_PART = 8388608
_N_BINS = 256
_LO, _HI = -6.0, 6.0
_BIN_W = (_HI - _LO) / _N_BINS
_INV_W = 1.0 / _BIN_W
_NOISE_SCALE = 0.05
_SEED = 0

_NC, _NS, _L = 2, 16, 16
_NW = _NC * _NS
_PER_W = _N_PART // _NW
_CHUNK = 32768
_NCHUNK = _PER_W // _CHUNK
_HC = _N_BINS + 1
_HW = _L * _HC

_mesh = plsc.VectorSubcoreMesh(core_axis_name="c", subcore_axis_name="s")


@functools.partial(
    pl.kernel,
    out_type=jax.ShapeDtypeStruct((_NW, _HW), jnp.float32),
    mesh=_mesh,
    compiler_params=pltpu.CompilerParams(
        needs_layout_passes=False,
        use_tc_tiling_on_sc=True,
    ),
    scratch_types=[
        pltpu.VMEM((_CHUNK,), jnp.float32),
        pltpu.VMEM((_CHUNK,), jnp.float32),
        pltpu.VMEM((_HW,), jnp.float32),
        pltpu.SemaphoreType.DMA,
        pltpu.SemaphoreType.DMA,
    ],
)
def _hist_sc(xp, zeros_hbm, out, buf0, buf1, hist, sem0, sem1):
    cid = lax.axis_index("c")
    sid = lax.axis_index("s")
    wid = sid * _NC + cid
    base = wid * _PER_W

    pltpu.sync_copy(zeros_hbm, hist)

    lane = lax.iota(jnp.int32, _L)
    laneoff = (lane * _HC).astype(jnp.float32)

    bufs = (buf0, buf1)
    sems = (sem0, sem1)

    def start(ci, b):
        pltpu.async_copy(
            xp.at[0, pl.ds(base + ci * _CHUNK, _CHUNK)], bufs[b], sems[b]
        )

    def wait(b):
        pltpu.make_async_copy(
            xp.at[0, pl.ds(base, _CHUNK)], bufs[b], sems[b]
        ).wait()

    def process(b):
        buf = bufs[b]

        @plsc.parallel_loop(0, _CHUNK // _L, unroll=8)
        def body(it):
            v = buf[pl.ds(it * _L, _L)]
            t = v * _INV_W + (-_LO * _INV_W)
            u = jnp.minimum(jnp.maximum(t, 0.0), float(_N_BINS - 1)) + laneoff
            i0 = u.astype(jnp.int32)
            f = u - i0.astype(jnp.float32)
            plsc.addupdate_scatter(hist, [i0], 1.0 - f)
            plsc.addupdate_scatter(hist, [i0 + 1], f)

    start(0, 0)
    start(1, 1)

    def outer(g, carry):
        for b in range(2):
            wait(b)
            process(b)

            @pl.when(g * 2 + b + 2 < _NCHUNK)
            def _():
                start(g * 2 + b + 2, b)

        return carry

    lax.fori_loop(0, _NCHUNK // 2, outer, 0)

    pltpu.sync_copy(hist, out.at[wid])


def _finish_body(parts_ref, scale_ref, o_ref):
    s = jnp.sum(parts_ref[...], axis=0)
    o_ref[...] = jnp.maximum(s * scale_ref[...], 0.0)


_finish = pl.pallas_call(
    _finish_body,
    out_shape=jax.ShapeDtypeStruct((_HC,), jnp.float32),
)


def kernel(x):
    xt = x.T
    zeros = jnp.zeros((_HW,), jnp.float32)
    parts = _hist_sc(xt, zeros)
    noise = (
        jax.random.normal(jax.random.key(_SEED), (_N_BINS,), jnp.float32)
        * _NOISE_SCALE
    )
    scale = jnp.concatenate(
        [(1.0 + noise) / (_N_PART * _BIN_W), jnp.zeros((1,), jnp.float32)]
    )
    out = _finish(parts.reshape(_NW * _L, _HC), scale)
    return out[:_N_BINS]
